# Initial kernel scaffold; baseline (speedup 1.0000x reference)
import jax
import jax.numpy as jnp
from jax.experimental import pallas as pl


def kernel(A, B):
    raise NotImplementedError("write your pallas kernel here")


# speedup vs baseline: 20.1085x; 20.1085x over previous
import math

import jax
import jax.numpy as jnp
from jax import lax
from jax.experimental import pallas as pl

PHASE_BINS = 100
EPS = 1e-10
W_AMP = 0.5
W_PHASE = 0.5
_PI = math.pi
_WIDTH = 2.0 * math.pi / PHASE_BINS
_N = 512 * 512


def _sample_body(a_ref, b_ref, amp_ref, ca_ref, cb_ref):
    a_re = a_ref[0, 0]
    a_im = a_ref[0, 1]
    b_re = b_ref[0, 0]
    b_im = b_ref[0, 1]

    a_abs = jnp.sqrt(a_re * a_re + a_im * a_im)
    b_abs = jnp.sqrt(b_re * b_re + b_im * b_im)
    s_a = jnp.sum(a_abs)
    s_b = jnp.sum(b_abs)
    p = a_abs / (s_a + EPS)
    q = b_abs / (s_b + EPS)
    amp = jnp.sum(p * jnp.log((p + EPS) / (q + EPS)))
    amp_ref[...] = jnp.full((1, 1, 128), amp, jnp.float32)

    lanes = lax.broadcasted_iota(jnp.int32, (1, 1, 128), 2)
    for ref, re, im in ((ca_ref, a_re, a_im), (cb_ref, b_re, b_im)):
        t = (jnp.arctan2(im, re) + _PI) / _WIDTH
        idxf = jnp.clip(jnp.floor(t), 0.0, PHASE_BINS - 1.0)
        acc = jnp.zeros((1, 1, 128), jnp.float32)
        for b in range(PHASE_BINS):
            cnt = jnp.sum(jnp.where(idxf == float(b), 1.0, 0.0))
            acc = acc + jnp.where(lanes == b, cnt, 0.0)
        ref[...] = acc


def _combine_body(amp_ref, ca_ref, cb_ref, out_ref):
    pp = ca_ref[:, 0, :] / (_N * _WIDTH)
    qp = cb_ref[:, 0, :] / (_N * _WIDTH)
    pp = pp / (jnp.sum(pp, axis=1, keepdims=True) + EPS)
    qp = qp / (jnp.sum(qp, axis=1, keepdims=True) + EPS)
    phase = jnp.sum(pp * jnp.log((pp + EPS) / (qp + EPS)))
    out_ref[...] = W_AMP * amp_ref[:, 0, :] + W_PHASE * phase


def kernel(A, B):
    amp, ca, cb = pl.pallas_call(
        _sample_body,
        grid=(32,),
        in_specs=[
            pl.BlockSpec((1, 2, 512, 512), lambda i: (i, 0, 0, 0)),
            pl.BlockSpec((1, 2, 512, 512), lambda i: (i, 0, 0, 0)),
        ],
        out_specs=[
            pl.BlockSpec((1, 1, 128), lambda i: (i, 0, 0)),
            pl.BlockSpec((1, 1, 128), lambda i: (i, 0, 0)),
            pl.BlockSpec((1, 1, 128), lambda i: (i, 0, 0)),
        ],
        out_shape=[
            jax.ShapeDtypeStruct((32, 1, 128), jnp.float32),
            jax.ShapeDtypeStruct((32, 1, 128), jnp.float32),
            jax.ShapeDtypeStruct((32, 1, 128), jnp.float32),
        ],
    )(A, B)
    out = pl.pallas_call(
        _combine_body,
        out_shape=jax.ShapeDtypeStruct((32, 128), jnp.float32),
    )(amp, ca, cb)
    return out[:, 0]


# speedup vs baseline: 39.8073x; 1.9796x over previous
import functools
import math

import jax
import jax.numpy as jnp
from jax import lax
from jax.experimental import pallas as pl
from jax.experimental.pallas import tpu as pltpu
from jax.experimental.pallas import tpu_sc as plsc

PHASE_BINS = 100
EPS = 1e-10
W_AMP = 0.5
W_PHASE = 0.5
_PI = math.pi
_WIDTH = 2.0 * math.pi / PHASE_BINS
_N = 512 * 512

_NC = 2
_NS = 16


def _sample_body(a_ref, b_ref, amp_ref, ia_ref, ib_ref):
    a_re = a_ref[0, 0]
    a_im = a_ref[0, 1]
    b_re = b_ref[0, 0]
    b_im = b_ref[0, 1]

    a_abs = jnp.sqrt(a_re * a_re + a_im * a_im)
    b_abs = jnp.sqrt(b_re * b_re + b_im * b_im)
    s_a = jnp.sum(a_abs)
    s_b = jnp.sum(b_abs)
    p = a_abs / (s_a + EPS)
    q = b_abs / (s_b + EPS)
    amp = jnp.sum(p * jnp.log((p + EPS) / (q + EPS)))
    amp_ref[...] = jnp.full((1, 1, 128), amp, jnp.float32)

    for ref, re, im in ((ia_ref, a_re, a_im), (ib_ref, b_re, b_im)):
        t = (jnp.arctan2(im, re) + _PI) / _WIDTH
        idx = jnp.clip(jnp.floor(t), 0.0, PHASE_BINS - 1.0).astype(jnp.int32)
        for k in range(4):
            ref[0, k] = idx[:, 128 * k:128 * (k + 1)]


def _sc_hist_body(ia_hbm, ib_hbm, oa_hbm, ob_hbm, buf_ref, hist_ref):
    row = lax.axis_index("s") * _NC + lax.axis_index("c")
    ones = jnp.full((16,), 1.0, jnp.float32)
    for idx_hbm, out_hbm in ((ia_hbm, oa_hbm), (ib_hbm, ob_hbm)):
        for j in range(8):
            hist_ref[pl.ds(16 * j, 16)] = jnp.zeros((16,), jnp.float32)

        def chunk(c, _, idx_hbm=idx_hbm):
            k = c // 2
            rs = (c % 2) * 256
            pltpu.sync_copy(idx_hbm.at[row, k, pl.ds(rs, 256), :], buf_ref)

            def inner(i, _):
                for j in range(8):
                    v = buf_ref[i, pl.ds(16 * j, 16)]
                    plsc.addupdate_scatter(hist_ref, [v], ones)
                return 0

            lax.fori_loop(0, 256, inner, 0)
            return 0

        lax.fori_loop(0, 8, chunk, 0)
        pltpu.sync_copy(hist_ref, out_hbm.at[row])


_sc_hist = functools.partial(
    pl.kernel,
    out_type=[
        jax.ShapeDtypeStruct((32, 128), jnp.float32),
        jax.ShapeDtypeStruct((32, 128), jnp.float32),
    ],
    mesh=plsc.VectorSubcoreMesh(core_axis_name="c", subcore_axis_name="s"),
    compiler_params=pltpu.CompilerParams(needs_layout_passes=False),
    scratch_types=[
        pltpu.VMEM((256, 128), jnp.int32),
        pltpu.VMEM((128,), jnp.float32),
    ],
)(_sc_hist_body)


def _combine_body(amp_ref, ca_ref, cb_ref, out_ref):
    pp = ca_ref[...] / (_N * _WIDTH)
    qp = cb_ref[...] / (_N * _WIDTH)
    pp = pp / (jnp.sum(pp, axis=1, keepdims=True) + EPS)
    qp = qp / (jnp.sum(qp, axis=1, keepdims=True) + EPS)
    phase = jnp.sum(pp * jnp.log((pp + EPS) / (qp + EPS)))
    out_ref[...] = W_AMP * amp_ref[:, 0, :] + W_PHASE * phase


def kernel(A, B):
    amp, ia, ib = pl.pallas_call(
        _sample_body,
        grid=(32,),
        in_specs=[
            pl.BlockSpec((1, 2, 512, 512), lambda i: (i, 0, 0, 0)),
            pl.BlockSpec((1, 2, 512, 512), lambda i: (i, 0, 0, 0)),
        ],
        out_specs=[
            pl.BlockSpec((1, 1, 128), lambda i: (i, 0, 0)),
            pl.BlockSpec((1, 4, 512, 128), lambda i: (i, 0, 0, 0)),
            pl.BlockSpec((1, 4, 512, 128), lambda i: (i, 0, 0, 0)),
        ],
        out_shape=[
            jax.ShapeDtypeStruct((32, 1, 128), jnp.float32),
            jax.ShapeDtypeStruct((32, 4, 512, 128), jnp.int32),
            jax.ShapeDtypeStruct((32, 4, 512, 128), jnp.int32),
        ],
    )(A, B)
    ca, cb = _sc_hist(ia, ib)
    out = pl.pallas_call(
        _combine_body,
        out_shape=jax.ShapeDtypeStruct((32, 128), jnp.float32),
    )(amp, ca, cb)
    return out[:, 0]


# speedup vs baseline: 64.1596x; 1.6118x over previous
import functools
import math

import jax
import jax.numpy as jnp
from jax import lax
from jax.experimental import pallas as pl
from jax.experimental.pallas import tpu as pltpu
from jax.experimental.pallas import tpu_sc as plsc

PHASE_BINS = 100
EPS = 1e-10
W_AMP = 0.5
W_PHASE = 0.5
_PI = math.pi
_WIDTH = 2.0 * math.pi / PHASE_BINS
_N = 512 * 512

_NC = 2
_NS = 16


def _sample_body(a_ref, b_ref, amp_ref, ia_ref, ib_ref):
    a_re = a_ref[0, 0]
    a_im = a_ref[0, 1]
    b_re = b_ref[0, 0]
    b_im = b_ref[0, 1]

    a_abs = jnp.sqrt(a_re * a_re + a_im * a_im)
    b_abs = jnp.sqrt(b_re * b_re + b_im * b_im)
    s_a = jnp.sum(a_abs)
    s_b = jnp.sum(b_abs)
    p = a_abs / (s_a + EPS)
    q = b_abs / (s_b + EPS)
    amp = jnp.sum(p * jnp.log((p + EPS) / (q + EPS)))
    amp_ref[...] = jnp.full((1, 1, 128), amp, jnp.float32)

    for ref, re, im in ((ia_ref, a_re, a_im), (ib_ref, b_re, b_im)):
        t = (jnp.arctan2(im, re) + _PI) / _WIDTH
        idx = jnp.clip(jnp.floor(t), 0.0, PHASE_BINS - 1.0).astype(jnp.int32)
        for k in range(4):
            ref[0, k] = idx[:, 128 * k:128 * (k + 1)]


def _sc_hist_body(ia_hbm, ib_hbm, oa_hbm, ob_hbm, buf0, buf1, hist_ref,
                  out_v, sem0, sem1):
    row = lax.axis_index("s") * _NC + lax.axis_index("c")
    ones = jnp.full((16,), 1.0, jnp.float32)
    zeros = jnp.zeros((16,), jnp.float32)
    bufs = (buf0, buf1)
    sems = (sem0, sem1)
    for idx_hbm, out_hbm in ((ia_hbm, oa_hbm), (ib_hbm, ob_hbm)):
        for j in range(64):
            hist_ref[pl.ds(16 * j, 16)] = zeros

        def start(c, idx_hbm=idx_hbm):
            k = c // 2
            rs = (c % 2) * 256
            return pltpu.async_copy(
                idx_hbm.at[row, k, pl.ds(rs, 256), :], bufs[c % 2], sems[c % 2])

        cp = start(0)
        for c in range(8):
            cp.wait()
            if c < 7:
                cp = start(c + 1)
            buf = bufs[c % 2]

            @plsc.parallel_loop(0, 256)
            def _(i, buf=buf):
                for j in range(8):
                    v = buf[i, pl.ds(16 * j, 16)]
                    plsc.addupdate_scatter(hist_ref, [v + 128 * j], ones)

        for lb in range(8):
            acc = hist_ref[pl.ds(16 * lb, 16)]
            for j in range(1, 8):
                acc = acc + hist_ref[pl.ds(128 * j + 16 * lb, 16)]
            out_v[pl.ds(16 * lb, 16)] = acc
        pltpu.sync_copy(out_v, out_hbm.at[row])


_sc_hist = functools.partial(
    pl.kernel,
    out_type=[
        jax.ShapeDtypeStruct((32, 128), jnp.float32),
        jax.ShapeDtypeStruct((32, 128), jnp.float32),
    ],
    mesh=plsc.VectorSubcoreMesh(core_axis_name="c", subcore_axis_name="s"),
    compiler_params=pltpu.CompilerParams(needs_layout_passes=False),
    scratch_types=[
        pltpu.VMEM((256, 128), jnp.int32),
        pltpu.VMEM((256, 128), jnp.int32),
        pltpu.VMEM((1024,), jnp.float32),
        pltpu.VMEM((128,), jnp.float32),
        pltpu.SemaphoreType.DMA,
        pltpu.SemaphoreType.DMA,
    ],
)(_sc_hist_body)


def _combine_body(amp_ref, ca_ref, cb_ref, out_ref):
    pp = ca_ref[...] / (_N * _WIDTH)
    qp = cb_ref[...] / (_N * _WIDTH)
    pp = pp / (jnp.sum(pp, axis=1, keepdims=True) + EPS)
    qp = qp / (jnp.sum(qp, axis=1, keepdims=True) + EPS)
    phase = jnp.sum(pp * jnp.log((pp + EPS) / (qp + EPS)))
    out_ref[...] = W_AMP * amp_ref[:, 0, :] + W_PHASE * phase


def kernel(A, B):
    amp, ia, ib = pl.pallas_call(
        _sample_body,
        grid=(32,),
        in_specs=[
            pl.BlockSpec((1, 2, 512, 512), lambda i: (i, 0, 0, 0)),
            pl.BlockSpec((1, 2, 512, 512), lambda i: (i, 0, 0, 0)),
        ],
        out_specs=[
            pl.BlockSpec((1, 1, 128), lambda i: (i, 0, 0)),
            pl.BlockSpec((1, 4, 512, 128), lambda i: (i, 0, 0, 0)),
            pl.BlockSpec((1, 4, 512, 128), lambda i: (i, 0, 0, 0)),
        ],
        out_shape=[
            jax.ShapeDtypeStruct((32, 1, 128), jnp.float32),
            jax.ShapeDtypeStruct((32, 4, 512, 128), jnp.int32),
            jax.ShapeDtypeStruct((32, 4, 512, 128), jnp.int32),
        ],
    )(A, B)
    ca, cb = _sc_hist(ia, ib)
    out = pl.pallas_call(
        _combine_body,
        out_shape=jax.ShapeDtypeStruct((32, 128), jnp.float32),
    )(amp, ca, cb)
    return out[:, 0]


# speedup vs baseline: 93.1133x; 1.4513x over previous
import functools
import math

import jax
import jax.numpy as jnp
from jax import lax
from jax.experimental import pallas as pl
from jax.experimental.pallas import tpu as pltpu
from jax.experimental.pallas import tpu_sc as plsc

PHASE_BINS = 100
EPS = 1e-10
W_AMP = 0.5
W_PHASE = 0.5
_PI = math.pi
_WIDTH = 2.0 * math.pi / PHASE_BINS
_N = 512 * 512

_NC = 2
_NS = 16


def _amp_body(a_ref, b_ref, amp_ref):
    a_re = a_ref[0, 0]
    a_im = a_ref[0, 1]
    b_re = b_ref[0, 0]
    b_im = b_ref[0, 1]

    a_abs = jnp.sqrt(a_re * a_re + a_im * a_im)
    b_abs = jnp.sqrt(b_re * b_re + b_im * b_im)
    s_a = jnp.sum(a_abs)
    s_b = jnp.sum(b_abs)
    ra = 1.0 / (s_a + EPS)
    rb = 1.0 / (s_b + EPS)
    ca = EPS * (s_a + EPS)
    cb = EPS * (s_b + EPS)
    la = jnp.log(a_abs + ca)
    lb = jnp.log(b_abs + cb)
    amp = ra * jnp.sum(a_abs * (la - lb)) + (ra * s_a) * (
        jnp.log(ra) - jnp.log(rb))
    amp_ref[...] = jnp.full((1, 1, 128), amp, jnp.float32)


_S = 50.0 / _PI
_ATAN_C = tuple(
    c * _S
    for c in (
        0.9999980168753235,
        -0.33306016681446504,
        0.19605492463877072,
        -0.12227066189358672,
        0.05855974329099421,
        -0.013887622675850114,
    )
)


def _phase_bin(re, im):
    ax = jnp.abs(re)
    ay = jnp.abs(im)
    e_m = ay > ax
    sx_m = re < 0.0
    sy_m = im < 0.0
    hi = jnp.maximum(ax, ay)
    lo = jnp.minimum(ax, ay)
    t = lo * pl.reciprocal(hi + 1e-20, approx=True, full_range=False)
    u = t * t
    p = jnp.float32(_ATAN_C[5])
    for c in _ATAN_C[4::-1]:
        p = p * u + jnp.float32(c)
    v = t * p
    g = jnp.where(sy_m, -1.0, 1.0)
    h = jnp.where(sx_m, -1.0, 1.0)
    s1 = jnp.where(e_m, -1.0, 1.0)
    gh = g * h
    big_g = gh * s1
    big_k = 50.0 + jnp.where(sx_m, g * 50.0, 0.0) + jnp.where(e_m, gh * 25.0, 0.0)
    idxi = (big_k + big_g * v).astype(jnp.int32)
    return jnp.clip(idxi, 0, PHASE_BINS - 1)


def _idx_body(a_ref, b_ref, ia_ref, ib_ref):
    for ref, src in ((ia_ref, a_ref), (ib_ref, b_ref)):
        idx = _phase_bin(src[0, 0], src[0, 1])
        for k in range(4):
            ref[0, k] = idx[:, 128 * k:128 * (k + 1)]


def _sc_accumulate(idx_hbm, out_hbm, row, bufs, hist_ref, out_v, sems):
    ones = jnp.full((16,), 1.0, jnp.float32)
    zeros = jnp.zeros((16,), jnp.float32)
    for j in range(64):
        hist_ref[pl.ds(16 * j, 16)] = zeros

    def start(c):
        k = c // 2
        rs = (c % 2) * 256
        return pltpu.async_copy(
            idx_hbm.at[row, k, pl.ds(rs, 256), :], bufs[c % 2], sems[c % 2])

    cp = start(0)
    for c in range(8):
        cp.wait()
        if c < 7:
            cp = start(c + 1)
        buf = bufs[c % 2]

        @plsc.parallel_loop(0, 256)
        def _(i, buf=buf):
            for j in range(8):
                v = buf[i, pl.ds(16 * j, 16)]
                plsc.addupdate_scatter(hist_ref, [v + 128 * j], ones)

    for lb in range(8):
        acc = hist_ref[pl.ds(16 * lb, 16)]
        for j in range(1, 8):
            acc = acc + hist_ref[pl.ds(128 * j + 16 * lb, 16)]
        out_v[pl.ds(16 * lb, 16)] = acc
    pltpu.sync_copy(out_v, out_hbm.at[row])


def _sc_hist_body(ia_hbm, ib_hbm, oa_hbm, ob_hbm, buf0, buf1, hist_ref,
                  out_v, sem0, sem1):
    wid = lax.axis_index("s") * _NC + lax.axis_index("c")
    row = lax.rem(wid, 16)
    is_b = wid >= 16
    bufs = (buf0, buf1)
    sems = (sem0, sem1)

    @pl.when(jnp.logical_not(is_b))
    def _():
        _sc_accumulate(ia_hbm, oa_hbm, row, bufs, hist_ref, out_v, sems)

    @pl.when(is_b)
    def _():
        _sc_accumulate(ib_hbm, ob_hbm, row, bufs, hist_ref, out_v, sems)


@functools.lru_cache(maxsize=None)
def _make_sc_hist():
    return pl.kernel(
        _sc_hist_body,
        out_type=[
            jax.ShapeDtypeStruct((16, 128), jnp.float32),
            jax.ShapeDtypeStruct((16, 128), jnp.float32),
        ],
        mesh=plsc.VectorSubcoreMesh(core_axis_name="c", subcore_axis_name="s"),
        compiler_params=pltpu.CompilerParams(needs_layout_passes=False),
        scratch_types=[
            pltpu.VMEM((256, 128), jnp.int32),
            pltpu.VMEM((256, 128), jnp.int32),
            pltpu.VMEM((1024,), jnp.float32),
            pltpu.VMEM((128,), jnp.float32),
            pltpu.SemaphoreType.DMA,
            pltpu.SemaphoreType.DMA,
        ],
    )


def _combine_body(amp_ref, ca0_ref, ca1_ref, cb0_ref, cb1_ref, out_ref):
    ca = jnp.concatenate([ca0_ref[...], ca1_ref[...]], axis=0)
    cb = jnp.concatenate([cb0_ref[...], cb1_ref[...]], axis=0)
    pp = ca / (_N * _WIDTH)
    qp = cb / (_N * _WIDTH)
    pp = pp / (jnp.sum(pp, axis=1, keepdims=True) + EPS)
    qp = qp / (jnp.sum(qp, axis=1, keepdims=True) + EPS)
    phase = jnp.sum(pp * jnp.log((pp + EPS) / (qp + EPS)))
    out_ref[...] = W_AMP * amp_ref[:, 0, :] + W_PHASE * phase


def kernel(A, B):
    sc_hist = _make_sc_hist()
    counts = []
    for h in range(2):
        ia, ib = pl.pallas_call(
            _idx_body,
            grid=(16,),
            in_specs=[
                pl.BlockSpec((1, 2, 512, 512),
                             lambda i, h=h: (i + 16 * h, 0, 0, 0)),
                pl.BlockSpec((1, 2, 512, 512),
                             lambda i, h=h: (i + 16 * h, 0, 0, 0)),
            ],
            out_specs=[
                pl.BlockSpec((1, 4, 512, 128), lambda i: (i, 0, 0, 0)),
                pl.BlockSpec((1, 4, 512, 128), lambda i: (i, 0, 0, 0)),
            ],
            out_shape=[
                jax.ShapeDtypeStruct((16, 4, 512, 128), jnp.int32),
                jax.ShapeDtypeStruct((16, 4, 512, 128), jnp.int32),
            ],
        )(A, B)
        counts.append(sc_hist(ia, ib))
    (ca0, cb0), (ca1, cb1) = counts
    amp = pl.pallas_call(
        _amp_body,
        grid=(32,),
        in_specs=[
            pl.BlockSpec((1, 2, 512, 512), lambda i: (i, 0, 0, 0)),
            pl.BlockSpec((1, 2, 512, 512), lambda i: (i, 0, 0, 0)),
        ],
        out_specs=[
            pl.BlockSpec((1, 1, 128), lambda i: (i, 0, 0)),
        ],
        out_shape=[
            jax.ShapeDtypeStruct((32, 1, 128), jnp.float32),
        ],
    )(A, B)
    amp = amp[0]
    out = pl.pallas_call(
        _combine_body,
        out_shape=jax.ShapeDtypeStruct((32, 128), jnp.float32),
    )(amp, ca0, ca1, cb0, cb1)
    return out[:, 0]


# speedup vs baseline: 94.9102x; 1.0193x over previous
import functools
import math

import jax
import jax.numpy as jnp
from jax import lax
from jax.experimental import pallas as pl
from jax.experimental.pallas import tpu as pltpu
from jax.experimental.pallas import tpu_sc as plsc

PHASE_BINS = 100
EPS = 1e-10
W_AMP = 0.5
W_PHASE = 0.5
_PI = math.pi
_WIDTH = 2.0 * math.pi / PHASE_BINS
_N = 512 * 512

_NC = 2
_NS = 16


def _amp_body(a_ref, b_ref, amp_ref):
    a_re = a_ref[0, 0]
    a_im = a_ref[0, 1]
    b_re = b_ref[0, 0]
    b_im = b_ref[0, 1]

    a_abs = jnp.sqrt(a_re * a_re + a_im * a_im)
    b_abs = jnp.sqrt(b_re * b_re + b_im * b_im)
    s_a = jnp.sum(a_abs)
    s_b = jnp.sum(b_abs)
    ra = 1.0 / (s_a + EPS)
    rb = 1.0 / (s_b + EPS)
    ca = EPS * (s_a + EPS)
    cb = EPS * (s_b + EPS)
    la = jnp.log(a_abs + ca)
    lb = jnp.log(b_abs + cb)
    amp = ra * jnp.sum(a_abs * (la - lb)) + (ra * s_a) * (
        jnp.log(ra) - jnp.log(rb))
    amp_ref[...] = jnp.full((1, 1, 128), amp, jnp.float32)


_S = 50.0 / _PI
_ATAN_C = tuple(
    c * _S
    for c in (
        0.9999980168753235,
        -0.33306016681446504,
        0.19605492463877072,
        -0.12227066189358672,
        0.05855974329099421,
        -0.013887622675850114,
    )
)


def _phase_bin(re, im):
    ax = jnp.abs(re)
    ay = jnp.abs(im)
    e_m = ay > ax
    sx_m = re < 0.0
    sy_m = im < 0.0
    hi = jnp.maximum(ax, ay)
    lo = jnp.minimum(ax, ay)
    t = lo * pl.reciprocal(hi + 1e-20, approx=True, full_range=False)
    u = t * t
    p = jnp.float32(_ATAN_C[5])
    for c in _ATAN_C[4::-1]:
        p = p * u + jnp.float32(c)
    v = t * p
    sign = jnp.int32(-2147483648)
    sb = jnp.bitwise_and(
        jnp.bitwise_xor(lax.bitcast_convert_type(re, jnp.int32),
                        lax.bitcast_convert_type(im, jnp.int32)), sign)
    gs = jnp.bitwise_xor(sb, jnp.where(e_m, sign, jnp.int32(0)))
    gv = lax.bitcast_convert_type(
        jnp.bitwise_xor(lax.bitcast_convert_type(v, jnp.int32), gs),
        jnp.float32)
    k0 = jnp.where(sy_m, 0.0, 100.0)
    k1 = jnp.where(sx_m, k0, 50.0)
    k2 = jnp.where(sy_m, 25.0, 75.0)
    big_k = jnp.where(e_m, k2, k1)
    idxi = (big_k + gv).astype(jnp.int32)
    return jnp.clip(idxi, 0, PHASE_BINS - 1)


def _idx_body(a_ref, b_ref, ia_ref, ib_ref):
    for ref, src in ((ia_ref, a_ref), (ib_ref, b_ref)):
        idx = _phase_bin(src[0, 0], src[0, 1])
        for k in range(4):
            ref[0, k] = idx[:, 128 * k:128 * (k + 1)]


def _sc_accumulate(idx_hbm, out_hbm, row, bufs, hist_ref, out_v, sems):
    ones = jnp.full((16,), 1.0, jnp.float32)
    zeros = jnp.zeros((16,), jnp.float32)
    for j in range(64):
        hist_ref[pl.ds(16 * j, 16)] = zeros

    def start(c):
        k = c // 2
        rs = (c % 2) * 256
        return pltpu.async_copy(
            idx_hbm.at[row, k, pl.ds(rs, 256), :], bufs[c % 2], sems[c % 2])

    cp = start(0)
    for c in range(8):
        cp.wait()
        if c < 7:
            cp = start(c + 1)
        buf = bufs[c % 2]

        @plsc.parallel_loop(0, 256)
        def _(i, buf=buf):
            for j in range(8):
                v = buf[i, pl.ds(16 * j, 16)]
                plsc.addupdate_scatter(hist_ref, [v + 128 * j], ones)

    for lb in range(8):
        acc = hist_ref[pl.ds(16 * lb, 16)]
        for j in range(1, 8):
            acc = acc + hist_ref[pl.ds(128 * j + 16 * lb, 16)]
        out_v[pl.ds(16 * lb, 16)] = acc
    pltpu.sync_copy(out_v, out_hbm.at[row])


def _sc_hist_body(ia_hbm, ib_hbm, oa_hbm, ob_hbm, buf0, buf1, hist_ref,
                  out_v, sem0, sem1):
    wid = lax.axis_index("s") * _NC + lax.axis_index("c")
    row = lax.rem(wid, 16)
    is_b = wid >= 16
    bufs = (buf0, buf1)
    sems = (sem0, sem1)

    @pl.when(jnp.logical_not(is_b))
    def _():
        _sc_accumulate(ia_hbm, oa_hbm, row, bufs, hist_ref, out_v, sems)

    @pl.when(is_b)
    def _():
        _sc_accumulate(ib_hbm, ob_hbm, row, bufs, hist_ref, out_v, sems)


@functools.lru_cache(maxsize=None)
def _make_sc_hist():
    return pl.kernel(
        _sc_hist_body,
        out_type=[
            jax.ShapeDtypeStruct((16, 128), jnp.float32),
            jax.ShapeDtypeStruct((16, 128), jnp.float32),
        ],
        mesh=plsc.VectorSubcoreMesh(core_axis_name="c", subcore_axis_name="s"),
        compiler_params=pltpu.CompilerParams(needs_layout_passes=False),
        scratch_types=[
            pltpu.VMEM((256, 128), jnp.int32),
            pltpu.VMEM((256, 128), jnp.int32),
            pltpu.VMEM((1024,), jnp.float32),
            pltpu.VMEM((128,), jnp.float32),
            pltpu.SemaphoreType.DMA,
            pltpu.SemaphoreType.DMA,
        ],
    )


def _combine_body(amp_ref, ca0_ref, ca1_ref, cb0_ref, cb1_ref, out_ref):
    ca = jnp.concatenate([ca0_ref[...], ca1_ref[...]], axis=0)
    cb = jnp.concatenate([cb0_ref[...], cb1_ref[...]], axis=0)
    pp = ca / (_N * _WIDTH)
    qp = cb / (_N * _WIDTH)
    pp = pp / (jnp.sum(pp, axis=1, keepdims=True) + EPS)
    qp = qp / (jnp.sum(qp, axis=1, keepdims=True) + EPS)
    phase = jnp.sum(pp * jnp.log((pp + EPS) / (qp + EPS)))
    out_ref[...] = W_AMP * amp_ref[:, 0, :] + W_PHASE * phase


def kernel(A, B):
    sc_hist = _make_sc_hist()
    counts = []
    for h in range(2):
        ia, ib = pl.pallas_call(
            _idx_body,
            grid=(16,),
            in_specs=[
                pl.BlockSpec((1, 2, 512, 512),
                             lambda i, h=h: (i + 16 * h, 0, 0, 0)),
                pl.BlockSpec((1, 2, 512, 512),
                             lambda i, h=h: (i + 16 * h, 0, 0, 0)),
            ],
            out_specs=[
                pl.BlockSpec((1, 4, 512, 128), lambda i: (i, 0, 0, 0)),
                pl.BlockSpec((1, 4, 512, 128), lambda i: (i, 0, 0, 0)),
            ],
            out_shape=[
                jax.ShapeDtypeStruct((16, 4, 512, 128), jnp.int32),
                jax.ShapeDtypeStruct((16, 4, 512, 128), jnp.int32),
            ],
        )(A, B)
        counts.append(sc_hist(ia, ib))
    (ca0, cb0), (ca1, cb1) = counts
    amp = pl.pallas_call(
        _amp_body,
        grid=(32,),
        in_specs=[
            pl.BlockSpec((1, 2, 512, 512), lambda i: (i, 0, 0, 0)),
            pl.BlockSpec((1, 2, 512, 512), lambda i: (i, 0, 0, 0)),
        ],
        out_specs=[
            pl.BlockSpec((1, 1, 128), lambda i: (i, 0, 0)),
        ],
        out_shape=[
            jax.ShapeDtypeStruct((32, 1, 128), jnp.float32),
        ],
    )(A, B)
    amp = amp[0]
    out = pl.pallas_call(
        _combine_body,
        out_shape=jax.ShapeDtypeStruct((32, 128), jnp.float32),
    )(amp, ca0, ca1, cb0, cb1)
    return out[:, 0]
